# baseline (speedup 1.0000x reference)
import functools

import jax
import jax.numpy as jnp
from jax import lax
from jax.experimental import pallas as pl
from jax.experimental.pallas import tpu as pltpu
from jax.experimental.pallas import tpu_sc as plsc

_K2 = 17
_NW = 32
_IDX_N = 128 * 128
_IDX_PER_W = _IDX_N // _NW
_EMB_N = _K2 * 768
_EMB_PER_W = _EMB_N // _NW


def _sc_body(table_hbm, idx_hbm, emb_hbm, idx_v, emb_v):
    wid = lax.axis_index("s") * 2 + lax.axis_index("c")

    ebase = wid * _EMB_PER_W
    pltpu.sync_copy(table_hbm.at[pl.ds(ebase, _EMB_PER_W)], emb_v)

    lane = lax.iota(jnp.int32, 16)
    base0 = wid * _IDX_PER_W
    for v in range(_IDX_PER_W // 16):
        n = base0 + v * 16 + lane
        i = n >> 7
        j = n & 127
        d = i - j
        mag = jnp.minimum((jnp.abs(d) + 1) >> 1, 8)
        idx_v[pl.ds(v * 16, 16)] = i * _K2 + 8 + jnp.where(d < 0, -mag, mag)

    pltpu.sync_copy(idx_v, idx_hbm.at[pl.ds(base0, _IDX_PER_W)])
    pltpu.sync_copy(emb_v, emb_hbm.at[pl.ds(ebase, _EMB_PER_W)])


def kernel(input_length, device, table):
    mesh = plsc.VectorSubcoreMesh(core_axis_name="c", subcore_axis_name="s")
    run = functools.partial(
        pl.kernel,
        mesh=mesh,
        out_type=(
            jax.ShapeDtypeStruct((_IDX_N,), jnp.int32),
            jax.ShapeDtypeStruct((_EMB_N,), jnp.float32),
        ),
        scratch_types=[
            pltpu.VMEM((_IDX_PER_W,), jnp.int32),
            pltpu.VMEM((_EMB_PER_W,), jnp.float32),
        ],
    )(_sc_body)
    idx, emb_flat = run(table.reshape(-1))
    return idx, emb_flat.reshape(_K2, 12, 64)


# speedup vs baseline: 1.0277x; 1.0277x over previous
import functools

import jax
import jax.numpy as jnp
from jax import lax
from jax.experimental import pallas as pl
from jax.experimental.pallas import tpu as pltpu
from jax.experimental.pallas import tpu_sc as plsc

_K2 = 17
_NW = 32
_IDX_N = 128 * 128
_IDX_PER_W = _IDX_N // _NW
_EMB_N = _K2 * 768
_EMB_PER_W = _EMB_N // _NW


def _sc_body(table_hbm, idx_hbm, emb_hbm, idx_v, emb_v):
    wid = lax.axis_index("s") * 2 + lax.axis_index("c")

    ebase = wid * _EMB_PER_W
    pltpu.sync_copy(table_hbm.at[pl.ds(ebase, _EMB_PER_W)], emb_v)

    lane = lax.iota(jnp.int32, 16)
    base0 = wid * _IDX_PER_W

    def step(v, _):
        n = base0 + v * 16 + lane
        i = n >> 7
        j = n & 127
        d = i - j
        mag = jnp.minimum((jnp.abs(d) + 1) >> 1, 8)
        off = pl.multiple_of(v * 16, 16)
        idx_v[pl.ds(off, 16)] = i * _K2 + 8 + jnp.where(d < 0, -mag, mag)
        return 0

    lax.fori_loop(0, _IDX_PER_W // 16, step, 0)

    pltpu.sync_copy(idx_v, idx_hbm.at[pl.ds(base0, _IDX_PER_W)])
    pltpu.sync_copy(emb_v, emb_hbm.at[pl.ds(ebase, _EMB_PER_W)])


def kernel(input_length, device, table):
    mesh = plsc.VectorSubcoreMesh(core_axis_name="c", subcore_axis_name="s")
    run = functools.partial(
        pl.kernel,
        mesh=mesh,
        out_type=(
            jax.ShapeDtypeStruct((_IDX_N,), jnp.int32),
            jax.ShapeDtypeStruct((_EMB_N,), jnp.float32),
        ),
        scratch_types=[
            pltpu.VMEM((_IDX_PER_W,), jnp.int32),
            pltpu.VMEM((_EMB_PER_W,), jnp.float32),
        ],
    )(_sc_body)
    idx, emb_flat = run(table.reshape(-1))
    return idx, emb_flat.reshape(_K2, 12, 64)


# speedup vs baseline: 1.0355x; 1.0076x over previous
import functools

import jax
import jax.numpy as jnp
from jax import lax
from jax.experimental import pallas as pl
from jax.experimental.pallas import tpu as pltpu
from jax.experimental.pallas import tpu_sc as plsc

_K2 = 17
_NC = 1
_NW = 16 * _NC
_IDX_N = 128 * 128
_IDX_PER_W = _IDX_N // _NW
_EMB_N = _K2 * 768
_EMB_PER_W = _EMB_N // _NW


def _sc_body(table_hbm, idx_hbm, emb_hbm, idx_v, emb_v):
    wid = lax.axis_index("s") * _NC + lax.axis_index("c")

    ebase = wid * _EMB_PER_W
    pltpu.sync_copy(table_hbm.at[pl.ds(ebase, _EMB_PER_W)], emb_v)

    lane = lax.iota(jnp.int32, 16)
    base0 = wid * _IDX_PER_W

    def step(v, _):
        n = base0 + v * 16 + lane
        i = n >> 7
        j = n & 127
        d = i - j
        mag = jnp.minimum((jnp.abs(d) + 1) >> 1, 8)
        off = pl.multiple_of(v * 16, 16)
        idx_v[pl.ds(off, 16)] = i * _K2 + 8 + jnp.where(d < 0, -mag, mag)
        return 0

    lax.fori_loop(0, _IDX_PER_W // 16, step, 0)

    pltpu.sync_copy(idx_v, idx_hbm.at[pl.ds(base0, _IDX_PER_W)])
    pltpu.sync_copy(emb_v, emb_hbm.at[pl.ds(ebase, _EMB_PER_W)])


def kernel(input_length, device, table):
    mesh = plsc.VectorSubcoreMesh(
        core_axis_name="c", subcore_axis_name="s", num_cores=_NC)
    run = functools.partial(
        pl.kernel,
        mesh=mesh,
        out_type=(
            jax.ShapeDtypeStruct((_IDX_N,), jnp.int32),
            jax.ShapeDtypeStruct((_EMB_N,), jnp.float32),
        ),
        scratch_types=[
            pltpu.VMEM((_IDX_PER_W,), jnp.int32),
            pltpu.VMEM((_EMB_PER_W,), jnp.float32),
        ],
    )(_sc_body)
    idx, emb_flat = run(table.reshape(-1))
    return idx, emb_flat.reshape(_K2, 12, 64)


# speedup vs baseline: 1.0493x; 1.0134x over previous
import functools

import jax
import jax.numpy as jnp
from jax import lax
from jax.experimental import pallas as pl
from jax.experimental.pallas import tpu as pltpu
from jax.experimental.pallas import tpu_sc as plsc

_K2 = 17
_NC = 1
_NW = 16 * _NC
_IDX_N = 128 * 128
_IDX_PER_W = _IDX_N // _NW
_EMB_N = _K2 * 768
_EMB_PER_W = _EMB_N // _NW


def _sc_body(table_hbm, idx_hbm, emb_hbm, idx_v, emb_v, sem_in, sem_out):
    wid = lax.axis_index("s") * _NC + lax.axis_index("c")

    ebase = wid * _EMB_PER_W
    cp_in = pltpu.async_copy(table_hbm.at[pl.ds(ebase, _EMB_PER_W)],
                             emb_v, sem_in)

    lane = lax.iota(jnp.int32, 16)
    base0 = wid * _IDX_PER_W

    def step(v, _):
        n = base0 + v * 16 + lane
        i = n >> 7
        j = n & 127
        d = i - j
        mag = jnp.minimum((jnp.abs(d) + 1) >> 1, 8)
        off = pl.multiple_of(v * 16, 16)
        idx_v[pl.ds(off, 16)] = i * _K2 + 8 + jnp.where(d < 0, -mag, mag)
        return 0

    lax.fori_loop(0, _IDX_PER_W // 16, step, 0)

    cp_idx = pltpu.async_copy(idx_v, idx_hbm.at[pl.ds(base0, _IDX_PER_W)],
                              sem_out)
    cp_in.wait()
    cp_emb = pltpu.async_copy(emb_v, emb_hbm.at[pl.ds(ebase, _EMB_PER_W)],
                              sem_in)
    cp_idx.wait()
    cp_emb.wait()


def kernel(input_length, device, table):
    mesh = plsc.VectorSubcoreMesh(
        core_axis_name="c", subcore_axis_name="s", num_cores=_NC)
    run = functools.partial(
        pl.kernel,
        mesh=mesh,
        out_type=(
            jax.ShapeDtypeStruct((_IDX_N,), jnp.int32),
            jax.ShapeDtypeStruct((_EMB_N,), jnp.float32),
        ),
        scratch_types=[
            pltpu.VMEM((_IDX_PER_W,), jnp.int32),
            pltpu.VMEM((_EMB_PER_W,), jnp.float32),
            pltpu.SemaphoreType.DMA,
            pltpu.SemaphoreType.DMA,
        ],
    )(_sc_body)
    idx, emb_flat = run(table.reshape(-1))
    return idx, emb_flat.reshape(_K2, 12, 64)
